# baseline (speedup 1.0000x reference)
import functools

import jax
import jax.numpy as jnp
from jax import lax
from jax.experimental import pallas as pl
from jax.experimental.pallas import tpu as pltpu
from jax.experimental.pallas import tpu_sc as plsc

_B, _L, _T = 16, 32, 32
_D = 2048
_N = _B * _L * _T
_NC, _NS = 2, 16
_NW = _NC * _NS
_RPW = _N // _NW
_CH = 8
_NCHUNK = _RPW // _CH
_CPL = _T // _CH
_LPW = _RPW // _T
_LANES = 16


def _sc_embed(tokens_flat, emb_table, pos_table):
    mesh = plsc.VectorSubcoreMesh(core_axis_name="c", subcore_axis_name="s")

    @functools.partial(
        pl.kernel,
        out_type=jax.ShapeDtypeStruct((_N, _D), jnp.float32),
        mesh=mesh,
        scratch_types=[
            pltpu.VMEM((_RPW,), jnp.int32),
            pltpu.VMEM((_D,), jnp.float32),
            pltpu.VMEM((_D,), jnp.float32),
            pltpu.VMEM((_CH, _D), jnp.float32),
            pltpu.VMEM((_CH, _D), jnp.float32),
            pltpu.VMEM((_CH, _D), jnp.float32),
            pltpu.VMEM((_CH, _D), jnp.float32),
            pltpu.VMEM((_CH, _D), jnp.float32),
            pltpu.VMEM((_CH, _D), jnp.float32),
            pltpu.SemaphoreType.DMA,
            pltpu.SemaphoreType.DMA,
            pltpu.SemaphoreType.DMA,
            pltpu.SemaphoreType.DMA,
            pltpu.SemaphoreType.DMA,
            pltpu.SemaphoreType.DMA,
            pltpu.SemaphoreType.DMA,
            pltpu.SemaphoreType.DMA,
        ],
    )
    def k(tok_hbm, emb_hbm, pos_hbm, out_hbm, idx_v, posb0, posb1,
          gbuf0, gbuf1, sbuf0, sbuf1, sbuf2, sbuf3,
          psem0, psem1, gsem0, gsem1, ssem0, ssem1, ssem2, ssem3):
        posbs = (posb0, posb1)
        psems = (psem0, psem1)
        gbufs = (gbuf0, gbuf1)
        gsems = (gsem0, gsem1)
        sbufs = (sbuf0, sbuf1, sbuf2, sbuf3)
        ssems = (ssem0, ssem1, ssem2, ssem3)

        wid = lax.axis_index("s") * _NC + lax.axis_index("c")
        base = wid * _RPW
        l0 = (wid % 2) * _LPW
        pltpu.sync_copy(tok_hbm.at[pl.ds(base, _RPW)], idx_v)

        pltpu.async_copy(pos_hbm.at[l0], posbs[0], psems[0])
        for b in range(2):
            pltpu.async_copy(
                emb_hbm.at[idx_v.at[pl.ds(b * _CH, _CH)]], gbufs[b], gsems[b]
            )

        def lbody(i2, carry):
            for d in range(2):
                i = 2 * i2 + d
                pb = d

                pltpu.make_async_copy(
                    pos_hbm.at[l0], posbs[pb], psems[pb]
                ).wait()

                @pl.when(i + 1 < _LPW)
                def _():
                    pltpu.async_copy(
                        pos_hbm.at[l0 + i + 1], posbs[1 - pb], psems[1 - pb]
                    )

                for kk in range(_CPL):
                    c = i * _CPL + kk
                    gs = kk % 2
                    ss = kk

                    pltpu.make_async_copy(
                        emb_hbm.at[idx_v.at[pl.ds(0, _CH)]],
                        gbufs[gs], gsems[gs]
                    ).wait()

                    @pl.when(c >= _CPL)
                    def _():
                        pltpu.make_async_copy(
                            sbufs[ss], out_hbm.at[pl.ds(base, _CH)], ssems[ss]
                        ).wait()

                    @plsc.parallel_loop(0, _D // _LANES, unroll=4)
                    def _(j):
                        sl = pl.ds(j * _LANES, _LANES)
                        pv = posbs[pb][sl]
                        for r in range(_CH):
                            sbufs[ss][r, sl] = gbufs[gs][r, sl] + pv

                    pltpu.async_copy(
                        sbufs[ss], out_hbm.at[pl.ds(base + c * _CH, _CH)],
                        ssems[ss],
                    )

                    @pl.when(c + 2 < _NCHUNK)
                    def _():
                        pltpu.async_copy(
                            emb_hbm.at[idx_v.at[pl.ds((c + 2) * _CH, _CH)]],
                            gbufs[gs],
                            gsems[gs],
                        )

            return carry

        lax.fori_loop(0, _LPW // 2, lbody, 0)

        for ss in range(_CPL):
            pltpu.make_async_copy(
                sbufs[ss], out_hbm.at[pl.ds(base, _CH)], ssems[ss]
            ).wait()

    return k(tokens_flat, emb_table, pos_table)


def kernel(observations_tokens, emb_table, pos_table):
    tokens_flat = observations_tokens.reshape(_N).astype(jnp.int32)
    out = _sc_embed(tokens_flat, emb_table, pos_table)
    return out.reshape(_B, _L, _T, _D)


# speedup vs baseline: 1.0799x; 1.0799x over previous
import functools

import jax
import jax.numpy as jnp
from jax import lax
from jax.experimental import pallas as pl
from jax.experimental.pallas import tpu as pltpu
from jax.experimental.pallas import tpu_sc as plsc

_B, _L, _T = 16, 32, 32
_D = 2048
_N = _B * _L * _T
_NC, _NS = 2, 16
_NW = _NC * _NS
_RPW = _N // _NW
_CH = 8
_NCHUNK = _RPW // _CH
_CPL = _T // _CH
_LPW = _RPW // _T
_LANES = 16
_NG = 4
_NS_BUF = 2


def _sc_embed(tokens_flat, emb_table, pos_table):
    mesh = plsc.VectorSubcoreMesh(core_axis_name="c", subcore_axis_name="s")

    @functools.partial(
        pl.kernel,
        out_type=jax.ShapeDtypeStruct((_N, _D), jnp.float32),
        mesh=mesh,
        scratch_types=[
            pltpu.VMEM((_RPW,), jnp.int32),
            pltpu.VMEM((_D,), jnp.float32),
            pltpu.VMEM((_D,), jnp.float32),
            pltpu.VMEM((_CH, _D), jnp.float32),
            pltpu.VMEM((_CH, _D), jnp.float32),
            pltpu.VMEM((_CH, _D), jnp.float32),
            pltpu.VMEM((_CH, _D), jnp.float32),
            pltpu.VMEM((_CH, _D), jnp.float32),
            pltpu.VMEM((_CH, _D), jnp.float32),
            pltpu.SemaphoreType.DMA,
            pltpu.SemaphoreType.DMA,
            pltpu.SemaphoreType.DMA,
            pltpu.SemaphoreType.DMA,
            pltpu.SemaphoreType.DMA,
            pltpu.SemaphoreType.DMA,
            pltpu.SemaphoreType.DMA,
            pltpu.SemaphoreType.DMA,
        ],
    )
    def k(tok_hbm, emb_hbm, pos_hbm, out_hbm, idx_v, posb0, posb1,
          gbuf0, gbuf1, gbuf2, gbuf3, sbuf0, sbuf1,
          psem0, psem1, gsem0, gsem1, gsem2, gsem3, ssem0, ssem1):
        posbs = (posb0, posb1)
        psems = (psem0, psem1)
        gbufs = (gbuf0, gbuf1, gbuf2, gbuf3)
        gsems = (gsem0, gsem1, gsem2, gsem3)
        sbufs = (sbuf0, sbuf1)
        ssems = (ssem0, ssem1)

        wid = lax.axis_index("s") * _NC + lax.axis_index("c")
        base = wid * _RPW
        l0 = (wid % 2) * _LPW
        pltpu.sync_copy(tok_hbm.at[pl.ds(base, _RPW)], idx_v)

        pltpu.async_copy(pos_hbm.at[l0], posbs[0], psems[0])
        for b in range(_NG):
            pltpu.async_copy(
                emb_hbm.at[idx_v.at[pl.ds(b * _CH, _CH)]], gbufs[b], gsems[b]
            )

        def lbody(i2, carry):
            for d in range(2):
                i = 2 * i2 + d
                pb = d

                pltpu.make_async_copy(
                    pos_hbm.at[l0], posbs[pb], psems[pb]
                ).wait()

                @pl.when(i + 1 < _LPW)
                def _():
                    pltpu.async_copy(
                        pos_hbm.at[l0 + i + 1], posbs[1 - pb], psems[1 - pb]
                    )

                for kk in range(_CPL):
                    c = i * _CPL + kk
                    gs = kk % _NG if d == 0 else (kk + _CPL) % _NG
                    ss = (kk + d * _CPL) % _NS_BUF

                    pltpu.make_async_copy(
                        emb_hbm.at[idx_v.at[pl.ds(0, _CH)]],
                        gbufs[gs], gsems[gs]
                    ).wait()

                    @pl.when(c >= _NS_BUF)
                    def _():
                        pltpu.make_async_copy(
                            sbufs[ss], out_hbm.at[pl.ds(base, _CH)], ssems[ss]
                        ).wait()

                    @plsc.parallel_loop(0, _D // _LANES, unroll=4)
                    def _(j):
                        sl = pl.ds(j * _LANES, _LANES)
                        pv = posbs[pb][sl]
                        for r in range(_CH):
                            sbufs[ss][r, sl] = gbufs[gs][r, sl] + pv

                    pltpu.async_copy(
                        sbufs[ss], out_hbm.at[pl.ds(base + c * _CH, _CH)],
                        ssems[ss],
                    )

                    @pl.when(c + _NG < _NCHUNK)
                    def _():
                        pltpu.async_copy(
                            emb_hbm.at[idx_v.at[pl.ds((c + _NG) * _CH, _CH)]],
                            gbufs[gs],
                            gsems[gs],
                        )

            return carry

        lax.fori_loop(0, _LPW // 2, lbody, 0)

        for ss in range(_NS_BUF):
            pltpu.make_async_copy(
                sbufs[ss], out_hbm.at[pl.ds(base, _CH)], ssems[ss]
            ).wait()

    return k(tokens_flat, emb_table, pos_table)


def kernel(observations_tokens, emb_table, pos_table):
    tokens_flat = observations_tokens.reshape(_N).astype(jnp.int32)
    out = _sc_embed(tokens_flat, emb_table, pos_table)
    return out.reshape(_B, _L, _T, _D)
